# Initial kernel scaffold; baseline (speedup 1.0000x reference)
import jax
import jax.numpy as jnp
from jax.experimental import pallas as pl


def kernel(inputs, lf1_cache, lf2_cache, conv1_weight, conv2_weight, conv1_bias, conv2_bias, ln_weight, cu_seqlens):
    raise NotImplementedError("write your pallas kernel here")


# speedup vs baseline: 27.6339x; 27.6339x over previous
import functools

import jax
import jax.numpy as jnp
from jax.experimental import pallas as pl
from jax.experimental.pallas import tpu as pltpu

_B = 8
_TOTAL = 8192
_D = 512
_H = _D // 2
_EPS = 1e-6
_T = 512


def _lf_kernel(s_ref, x_ref, lf1_ref, lf2_ref, w1_ref, w2_ref, b1_ref,
               b2_ref, ln_ref, out_ref, lf1n_ref, lf2n_ref,
               carry1, carry2, cproj1, cproj2):
    i = pl.program_id(0)
    base = i * _T

    @pl.when(i == 0)
    def _init():
        cproj1[:] = jnp.dot(lf1_ref[:], w1_ref[:, :_H],
                            preferred_element_type=jnp.float32)
        cproj2[:] = jnp.dot(lf2_ref[:], w2_ref[:, :_D],
                            preferred_element_type=jnp.float32)
        lf1n_ref[:] = lf1_ref[:]
        lf2n_ref[:] = lf2_ref[:]
        carry1[:] = jnp.zeros_like(carry1)
        carry2[:] = jnp.zeros_like(carry2)

    x = x_ref[:]
    row = jax.lax.broadcasted_iota(jnp.int32, (_T, 1), 0)

    c1 = jnp.dot(x, w1_ref[:], preferred_element_type=jnp.float32)
    c1h = c1[:, :_H]
    prev1 = jnp.where(row == 0, carry1[:], pltpu.roll(c1h, 1, axis=0))
    for b in range(_B):
        prev1 = jnp.where(row == s_ref[b] - base, cproj1[b:b + 1, :], prev1)
    out1 = prev1 + c1[:, _H:] + b1_ref[:]
    carry1[:] = c1h[_T - 1:_T, :]

    c2 = jnp.dot(out1, w2_ref[:], preferred_element_type=jnp.float32)
    c2d = c2[:, :_D]
    prev2 = jnp.where(row == 0, carry2[:], pltpu.roll(c2d, 1, axis=0))
    for b in range(_B):
        prev2 = jnp.where(row == s_ref[b] - base, cproj2[b:b + 1, :], prev2)
    y = prev2 + c2[:, _D:] + b2_ref[:] + x
    carry2[:] = c2d[_T - 1:_T, :]

    var = jnp.mean(y * y, axis=-1, keepdims=True)
    out_ref[:] = y * jax.lax.rsqrt(var + _EPS) * ln_ref[:]

    for b in range(_B):
        lens_b = s_ref[b + 1] - s_ref[b]
        local = s_ref[b + 1] - 1 - base
        hit = (local >= 0) & (local < _T) & (lens_b > 0)

        @pl.when(hit)
        def _(b=b, local=local):
            mask = row == local
            lf1n_ref[b:b + 1, :] = jnp.sum(
                jnp.where(mask, x, 0.0), axis=0, keepdims=True)
            lf2n_ref[b:b + 1, :] = jnp.sum(
                jnp.where(mask, out1, 0.0), axis=0, keepdims=True)


@jax.jit
def kernel(inputs, lf1_cache, lf2_cache, conv1_weight, conv2_weight,
           conv1_bias, conv2_bias, ln_weight, cu_seqlens):
    lf1 = lf1_cache.reshape(_B, _D)
    lf2 = lf2_cache.reshape(_B, _H)
    b1 = conv1_bias.reshape(1, _H)
    b2 = conv2_bias.reshape(1, _D)
    ln = ln_weight.reshape(1, _D)
    n_tiles = _TOTAL // _T

    grid_spec = pltpu.PrefetchScalarGridSpec(
        num_scalar_prefetch=1,
        grid=(n_tiles,),
        in_specs=[
            pl.BlockSpec((_T, _D), lambda i, s: (i, 0)),
            pl.BlockSpec((_B, _D), lambda i, s: (0, 0)),
            pl.BlockSpec((_B, _H), lambda i, s: (0, 0)),
            pl.BlockSpec((_D, _D), lambda i, s: (0, 0)),
            pl.BlockSpec((_H, 2 * _D), lambda i, s: (0, 0)),
            pl.BlockSpec((1, _H), lambda i, s: (0, 0)),
            pl.BlockSpec((1, _D), lambda i, s: (0, 0)),
            pl.BlockSpec((1, _D), lambda i, s: (0, 0)),
        ],
        out_specs=[
            pl.BlockSpec((_T, _D), lambda i, s: (i, 0)),
            pl.BlockSpec((_B, _D), lambda i, s: (0, 0)),
            pl.BlockSpec((_B, _H), lambda i, s: (0, 0)),
        ],
        scratch_shapes=[
            pltpu.VMEM((1, _H), jnp.float32),
            pltpu.VMEM((1, _D), jnp.float32),
            pltpu.VMEM((_B, _H), jnp.float32),
            pltpu.VMEM((_B, _D), jnp.float32),
        ],
    )

    out, lf1n, lf2n = pl.pallas_call(
        _lf_kernel,
        grid_spec=grid_spec,
        out_shape=[
            jax.ShapeDtypeStruct((_TOTAL, _D), jnp.float32),
            jax.ShapeDtypeStruct((_B, _D), jnp.float32),
            jax.ShapeDtypeStruct((_B, _H), jnp.float32),
        ],
        compiler_params=pltpu.CompilerParams(
            dimension_semantics=("arbitrary",)),
    )(cu_seqlens, inputs, lf1, lf2, conv1_weight, conv2_weight, b1, b2, ln)

    return out, lf1n.reshape(_B, 1, _D), lf2n.reshape(_B, 1, _H)


# speedup vs baseline: 30.2684x; 1.0953x over previous
import jax
import jax.numpy as jnp
from jax.experimental import pallas as pl
from jax.experimental.pallas import tpu as pltpu

_B = 8
_TOTAL = 8192
_D = 512
_H = _D // 2
_EPS = 1e-6
_T = 512


def _lf_kernel(s_ref, x_ref, lf1_ref, lf2_ref, w1_ref, w2_ref, b1_ref,
               b2_ref, ln_ref, out_ref, lf1n_ref, lf2n_ref,
               carry1, carry2, cproj1, cproj2, o1_s, y_s):
    i = pl.program_id(0)
    base = i * _T

    @pl.when(i == 0)
    def _init():
        cproj1[:] = jnp.dot(lf1_ref[:], w1_ref[:, :_H],
                            preferred_element_type=jnp.float32)
        cproj2[:] = jnp.dot(lf2_ref[:], w2_ref[:, :_D],
                            preferred_element_type=jnp.float32)
        lf1n_ref[:] = lf1_ref[:]
        lf2n_ref[:] = lf2_ref[:]
        carry1[:] = jnp.zeros_like(carry1)
        carry2[:] = jnp.zeros_like(carry2)

    x = x_ref[:]
    row = jax.lax.broadcasted_iota(jnp.int32, (_T, 1), 0)

    c1 = jnp.dot(x, w1_ref[:], preferred_element_type=jnp.float32)
    c1h = c1[:, :_H]
    prev1 = jnp.where(row == 0, carry1[:], pltpu.roll(c1h, 1, axis=0))
    o1_s[:] = prev1 + c1[:, _H:] + b1_ref[:]
    carry1[:] = c1h[_T - 1:_T, :]

    for b in range(_B):
        local = s_ref[b] - base
        owns = s_ref[b + 1] > s_ref[b]

        @pl.when((local >= 0) & (local < _T) & owns)
        def _(b=b, local=local):
            xr = x_ref[pl.ds(local, 1), :]
            o1_s[pl.ds(local, 1), :] = (
                cproj1[b:b + 1, :]
                + jnp.dot(xr, w1_ref[:, _H:],
                          preferred_element_type=jnp.float32)
                + b1_ref[:])

    o1 = o1_s[:]
    c2 = jnp.dot(o1, w2_ref[:], preferred_element_type=jnp.float32)
    c2d = c2[:, :_D]
    prev2 = jnp.where(row == 0, carry2[:], pltpu.roll(c2d, 1, axis=0))
    y_s[:] = prev2 + c2[:, _D:] + b2_ref[:] + x
    carry2[:] = c2d[_T - 1:_T, :]

    for b in range(_B):
        local = s_ref[b] - base
        owns = s_ref[b + 1] > s_ref[b]

        @pl.when((local >= 0) & (local < _T) & owns)
        def _(b=b, local=local):
            o1r = o1_s[pl.ds(local, 1), :]
            y_s[pl.ds(local, 1), :] = (
                cproj2[b:b + 1, :]
                + jnp.dot(o1r, w2_ref[:, _D:],
                          preferred_element_type=jnp.float32)
                + b2_ref[:] + x_ref[pl.ds(local, 1), :])

    y = y_s[:]
    var = jnp.mean(y * y, axis=-1, keepdims=True)
    out_ref[:] = y * jax.lax.rsqrt(var + _EPS) * ln_ref[:]

    for b in range(_B):
        local = s_ref[b + 1] - 1 - base
        nonempty = s_ref[b + 1] > s_ref[b]

        @pl.when((local >= 0) & (local < _T) & nonempty)
        def _(b=b, local=local):
            lf1n_ref[b:b + 1, :] = x_ref[pl.ds(local, 1), :]
            lf2n_ref[b:b + 1, :] = o1_s[pl.ds(local, 1), :]


@jax.jit
def kernel(inputs, lf1_cache, lf2_cache, conv1_weight, conv2_weight,
           conv1_bias, conv2_bias, ln_weight, cu_seqlens):
    lf1 = lf1_cache.reshape(_B, _D)
    lf2 = lf2_cache.reshape(_B, _H)
    b1 = conv1_bias.reshape(1, _H)
    b2 = conv2_bias.reshape(1, _D)
    ln = ln_weight.reshape(1, _D)
    n_tiles = _TOTAL // _T

    grid_spec = pltpu.PrefetchScalarGridSpec(
        num_scalar_prefetch=1,
        grid=(n_tiles,),
        in_specs=[
            pl.BlockSpec((_T, _D), lambda i, s: (i, 0)),
            pl.BlockSpec((_B, _D), lambda i, s: (0, 0)),
            pl.BlockSpec((_B, _H), lambda i, s: (0, 0)),
            pl.BlockSpec((_D, _D), lambda i, s: (0, 0)),
            pl.BlockSpec((_H, 2 * _D), lambda i, s: (0, 0)),
            pl.BlockSpec((1, _H), lambda i, s: (0, 0)),
            pl.BlockSpec((1, _D), lambda i, s: (0, 0)),
            pl.BlockSpec((1, _D), lambda i, s: (0, 0)),
        ],
        out_specs=[
            pl.BlockSpec((_T, _D), lambda i, s: (i, 0)),
            pl.BlockSpec((_B, _D), lambda i, s: (0, 0)),
            pl.BlockSpec((_B, _H), lambda i, s: (0, 0)),
        ],
        scratch_shapes=[
            pltpu.VMEM((1, _H), jnp.float32),
            pltpu.VMEM((1, _D), jnp.float32),
            pltpu.VMEM((_B, _H), jnp.float32),
            pltpu.VMEM((_B, _D), jnp.float32),
            pltpu.VMEM((_T, _H), jnp.float32),
            pltpu.VMEM((_T, _D), jnp.float32),
        ],
    )

    out, lf1n, lf2n = pl.pallas_call(
        _lf_kernel,
        grid_spec=grid_spec,
        out_shape=[
            jax.ShapeDtypeStruct((_TOTAL, _D), jnp.float32),
            jax.ShapeDtypeStruct((_B, _D), jnp.float32),
            jax.ShapeDtypeStruct((_B, _H), jnp.float32),
        ],
        compiler_params=pltpu.CompilerParams(
            dimension_semantics=("arbitrary",)),
    )(cu_seqlens, inputs, lf1, lf2, conv1_weight, conv2_weight, b1, b2, ln)

    return out, lf1n.reshape(_B, 1, _D), lf2n.reshape(_B, 1, _H)
